# Initial kernel scaffold; baseline (speedup 1.0000x reference)
import jax
import jax.numpy as jnp
from jax.experimental import pallas as pl


def kernel(logprobs, beam_logprobs_sum, beam_seq, beam_seq_logprobs, state, beam_size):
    raise NotImplementedError("write your pallas kernel here")


# speedup vs baseline: 3.5509x; 3.5509x over previous
import functools

import jax
import jax.numpy as jnp
from jax import lax
from jax.experimental import pallas as pl
from jax.experimental.pallas import tpu as pltpu

BATCH, BDASH, VOCAB, T, DMODEL = 16, 8, 32768, 4, 1024
NEG_INF = float("-inf")


def _topk_body(lp_ref, bls_ref, seq_ref, bls_out, seq_out, src_out):
    b = pl.program_id(0)
    bls_col = bls_ref[0]
    cand = lp_ref[...] + bls_col
    fid = (lax.broadcasted_iota(jnp.int32, (BDASH, VOCAB), 0) * VOCAB
           + lax.broadcasted_iota(jnp.int32, (BDASH, VOCAB), 1))
    iota_col = lax.broadcasted_iota(jnp.int32, (BDASH, 1), 0)
    lane8 = lax.broadcasted_iota(jnp.int32, (1, 1, BDASH), 2)
    seq_slab = seq_ref[...]
    seq_mask_iota = lax.broadcasted_iota(jnp.int32, (1, BDASH, T), 1)

    nb = jnp.zeros((1, 1, BDASH), jnp.float32)
    sr = jnp.zeros((1, 1, BDASH), jnp.int32)
    vals = cand
    for k in range(BDASH):
        m = jnp.max(vals)
        idx = jnp.min(jnp.where(vals == m, fid, jnp.int32(2 ** 30)))
        beam_k = lax.shift_right_logical(idx, 15)
        sel_k = jnp.bitwise_and(idx, VOCAB - 1)
        blsv = jnp.sum(jnp.where(iota_col == beam_k, bls_col, 0.0))
        nb = jnp.where(lane8 == k, blsv + m, nb)
        sr = jnp.where(lane8 == k, b * BDASH + beam_k, sr)
        row = jnp.sum(jnp.where(seq_mask_iota == beam_k, seq_slab, 0),
                      axis=(0, 1))
        row5 = jnp.concatenate([row, jnp.broadcast_to(sel_k, (1,))], axis=0)
        seq_out[0, k, :] = row5
        vals = jnp.where(fid == idx, NEG_INF, vals)
    bls_out[...] = nb
    src_out[...] = sr


def _gather_body(src_ref, bsl_ref, lp_ref, st_ref, out_ref, ost_ref):
    del src_ref
    out_ref[0, 0:T, :] = bsl_ref[0]
    out_ref[0, T, :] = lp_ref[0, 0]
    ost_ref[...] = st_ref[...]


def kernel(logprobs, beam_logprobs_sum, beam_seq, beam_seq_logprobs, state,
           beam_size):
    del beam_size
    nrows = BATCH * BDASH

    new_bls3, new_seq, src3 = pl.pallas_call(
        _topk_body,
        grid=(BATCH,),
        in_specs=[
            pl.BlockSpec((BDASH, VOCAB), lambda b: (b, 0)),
            pl.BlockSpec((1, BDASH, 1), lambda b: (b, 0, 0)),
            pl.BlockSpec((1, BDASH, T), lambda b: (b, 0, 0)),
        ],
        out_specs=[
            pl.BlockSpec((1, 1, BDASH), lambda b: (b, 0, 0)),
            pl.BlockSpec((1, BDASH, T + 1), lambda b: (b, 0, 0)),
            pl.BlockSpec((1, 1, BDASH), lambda b: (b, 0, 0)),
        ],
        out_shape=[
            jax.ShapeDtypeStruct((BATCH, 1, BDASH), jnp.float32),
            jax.ShapeDtypeStruct((BATCH, BDASH, T + 1), jnp.int32),
            jax.ShapeDtypeStruct((BATCH, 1, BDASH), jnp.int32),
        ],
    )(
        logprobs.reshape(nrows, VOCAB),
        beam_logprobs_sum.reshape(BATCH, BDASH, 1),
        beam_seq,
    )

    src_rows = src3.reshape(nrows)

    bsl_flat = beam_seq_logprobs.reshape(nrows, T, VOCAB)
    lp3 = logprobs.reshape(nrows, 1, VOCAB)
    st4 = state.reshape(2, nrows, 1, DMODEL)

    new_bsl, new_state4 = pl.pallas_call(
        _gather_body,
        grid_spec=pltpu.PrefetchScalarGridSpec(
            num_scalar_prefetch=1,
            grid=(nrows,),
            in_specs=[
                pl.BlockSpec((1, T, VOCAB), lambda i, sr: (sr[i], 0, 0)),
                pl.BlockSpec((1, 1, VOCAB), lambda i, sr: (sr[i], 0, 0)),
                pl.BlockSpec((2, 1, 1, DMODEL), lambda i, sr: (0, sr[i], 0, 0)),
            ],
            out_specs=[
                pl.BlockSpec((1, T + 1, VOCAB), lambda i, sr: (i, 0, 0)),
                pl.BlockSpec((2, 1, 1, DMODEL), lambda i, sr: (0, i, 0, 0)),
            ],
        ),
        out_shape=[
            jax.ShapeDtypeStruct((nrows, T + 1, VOCAB), jnp.float32),
            jax.ShapeDtypeStruct((2, nrows, 1, DMODEL), jnp.float32),
        ],
    )(src_rows, bsl_flat, lp3, st4)

    new_beam_seq_logprobs = new_bsl.reshape(BATCH, BDASH, T + 1, VOCAB)
    new_beam_logprobs_sum = new_bls3.reshape(BATCH, BDASH)
    new_state = new_state4.reshape(2, nrows, DMODEL)
    return new_seq, new_beam_seq_logprobs, new_beam_logprobs_sum, new_state


# speedup vs baseline: 3.5795x; 1.0081x over previous
import jax
import jax.numpy as jnp
from jax import lax
from jax.experimental import pallas as pl
from jax.experimental.pallas import tpu as pltpu

BATCH, BDASH, VOCAB, T, DMODEL = 16, 8, 32768, 4, 1024
NEG_INF = float("-inf")


def _topk_body(lp_ref, bls_ref, seq_ref, st_ref,
               bls_out, seq_out, src_out, st_out):
    b = pl.program_id(0)
    bls_col = bls_ref[0]
    cand = lp_ref[...] + bls_col
    fid = (lax.broadcasted_iota(jnp.int32, (BDASH, VOCAB), 0) * VOCAB
           + lax.broadcasted_iota(jnp.int32, (BDASH, VOCAB), 1))
    iota_col = lax.broadcasted_iota(jnp.int32, (BDASH, 1), 0)
    lane8 = lax.broadcasted_iota(jnp.int32, (1, 1, BDASH), 2)
    seq_slab = seq_ref[...]
    seq_mask_iota = lax.broadcasted_iota(jnp.int32, (1, BDASH, T), 1)
    st_blk = st_ref[...]
    st_iota = lax.broadcasted_iota(jnp.int32, (2, BDASH, DMODEL), 1)

    nb = jnp.zeros((1, 1, BDASH), jnp.float32)
    sr = jnp.zeros((1, 1, BDASH), jnp.int32)
    vals = cand
    for k in range(BDASH):
        m = jnp.max(vals)
        idx = jnp.min(jnp.where(vals == m, fid, jnp.int32(2 ** 30)))
        beam_k = lax.shift_right_logical(idx, 15)
        sel_k = jnp.bitwise_and(idx, VOCAB - 1)
        blsv = jnp.sum(jnp.where(iota_col == beam_k, bls_col, 0.0))
        nb = jnp.where(lane8 == k, blsv + m, nb)
        sr = jnp.where(lane8 == k, b * BDASH + beam_k, sr)
        row = jnp.sum(jnp.where(seq_mask_iota == beam_k, seq_slab, 0),
                      axis=(0, 1))
        row5 = jnp.concatenate([row, jnp.broadcast_to(sel_k, (1,))], axis=0)
        seq_out[0, k, :] = row5
        st_out[:, k, :] = jnp.sum(
            jnp.where(st_iota == beam_k, st_blk, 0.0), axis=1)
        vals = jnp.where(fid == idx, NEG_INF, vals)
    bls_out[...] = nb
    src_out[...] = sr


def _gather_body(src_ref, bsl_ref, lp_ref, out_ref):
    i = pl.program_id(0)
    beam_k = src_ref[i] - (i // BDASH) * BDASH
    out_ref[0, 0, 0:T, :] = bsl_ref[0, 0]
    sub_iota = lax.broadcasted_iota(jnp.int32, (BDASH, VOCAB), 0)
    out_ref[0, 0, T, :] = jnp.sum(
        jnp.where(sub_iota == beam_k, lp_ref[0], 0.0), axis=0)


def kernel(logprobs, beam_logprobs_sum, beam_seq, beam_seq_logprobs, state,
           beam_size):
    del beam_size
    nrows = BATCH * BDASH

    new_bls3, new_seq, src3, new_state = pl.pallas_call(
        _topk_body,
        grid=(BATCH,),
        in_specs=[
            pl.BlockSpec((BDASH, VOCAB), lambda b: (b, 0)),
            pl.BlockSpec((1, BDASH, 1), lambda b: (b, 0, 0)),
            pl.BlockSpec((1, BDASH, T), lambda b: (b, 0, 0)),
            pl.BlockSpec((2, BDASH, DMODEL), lambda b: (0, b, 0)),
        ],
        out_specs=[
            pl.BlockSpec((1, 1, BDASH), lambda b: (b, 0, 0)),
            pl.BlockSpec((1, BDASH, T + 1), lambda b: (b, 0, 0)),
            pl.BlockSpec((1, 1, BDASH), lambda b: (b, 0, 0)),
            pl.BlockSpec((2, BDASH, DMODEL), lambda b: (0, b, 0)),
        ],
        out_shape=[
            jax.ShapeDtypeStruct((BATCH, 1, BDASH), jnp.float32),
            jax.ShapeDtypeStruct((BATCH, BDASH, T + 1), jnp.int32),
            jax.ShapeDtypeStruct((BATCH, 1, BDASH), jnp.int32),
            jax.ShapeDtypeStruct((2, nrows, DMODEL), jnp.float32),
        ],
    )(
        logprobs.reshape(nrows, VOCAB),
        beam_logprobs_sum.reshape(BATCH, BDASH, 1),
        beam_seq,
        state,
    )

    src_rows = src3.reshape(nrows)

    new_bsl = pl.pallas_call(
        _gather_body,
        grid_spec=pltpu.PrefetchScalarGridSpec(
            num_scalar_prefetch=1,
            grid=(nrows,),
            in_specs=[
                pl.BlockSpec((1, 1, T, VOCAB),
                             lambda i, sr: (sr[i] // BDASH, sr[i] % BDASH,
                                            0, 0)),
                pl.BlockSpec((1, BDASH, VOCAB),
                             lambda i, sr: (i // BDASH, 0, 0)),
            ],
            out_specs=[
                pl.BlockSpec((1, 1, T + 1, VOCAB),
                             lambda i, sr: (i // BDASH, i % BDASH, 0, 0)),
            ],
        ),
        out_shape=[
            jax.ShapeDtypeStruct((BATCH, BDASH, T + 1, VOCAB), jnp.float32),
        ],
    )(src_rows, beam_seq_logprobs, logprobs.reshape(BATCH, BDASH, VOCAB))[0]

    new_beam_logprobs_sum = new_bls3.reshape(BATCH, BDASH)
    return new_seq, new_bsl, new_beam_logprobs_sum, new_state


# speedup vs baseline: 3.6164x; 1.0103x over previous
import jax
import jax.numpy as jnp
from jax import lax
from jax.experimental import pallas as pl
from jax.experimental.pallas import tpu as pltpu

BATCH, BDASH, VOCAB, T, DMODEL = 16, 8, 32768, 4, 1024
NEG_INF = float("-inf")


def _topk_body(lp_ref, bls_ref, seq_ref, st_ref,
               bls_out, seq_out, src_out, st_out):
    b = pl.program_id(0)
    bls_col = bls_ref[0]
    cand = lp_ref[...] + bls_col
    fid = (lax.broadcasted_iota(jnp.int32, (BDASH, VOCAB), 0) * VOCAB
           + lax.broadcasted_iota(jnp.int32, (BDASH, VOCAB), 1))
    iota_col = lax.broadcasted_iota(jnp.int32, (BDASH, 1), 0)
    lane8 = lax.broadcasted_iota(jnp.int32, (1, 1, BDASH), 2)
    seq_slab = seq_ref[...]
    seq_mask_iota = lax.broadcasted_iota(jnp.int32, (1, BDASH, T), 1)
    st_blk = st_ref[...]
    st_iota = lax.broadcasted_iota(jnp.int32, (2, BDASH, DMODEL), 1)

    nb = jnp.zeros((1, 1, BDASH), jnp.float32)
    sr = jnp.zeros((1, 1, BDASH), jnp.int32)
    vals = cand
    for k in range(BDASH):
        m = jnp.max(vals)
        idx = jnp.min(jnp.where(vals == m, fid, jnp.int32(2 ** 30)))
        beam_k = lax.shift_right_logical(idx, 15)
        sel_k = jnp.bitwise_and(idx, VOCAB - 1)
        blsv = jnp.sum(jnp.where(iota_col == beam_k, bls_col, 0.0))
        nb = jnp.where(lane8 == k, blsv + m, nb)
        sr = jnp.where(lane8 == k, b * BDASH + beam_k, sr)
        row = jnp.sum(jnp.where(seq_mask_iota == beam_k, seq_slab, 0),
                      axis=(0, 1))
        row5 = jnp.concatenate([row, jnp.broadcast_to(sel_k, (1,))], axis=0)
        seq_out[0, k, :] = row5
        st_out[:, k, :] = jnp.sum(
            jnp.where(st_iota == beam_k, st_blk, 0.0), axis=1)
        vals = jnp.where(fid == idx, NEG_INF, vals)
    bls_out[...] = nb
    src_out[...] = sr


def _gather_body(src_ref, bsl_ref, lp_ref, out_ref):
    i = pl.program_id(0)
    beam_k = src_ref[i] - (i // BDASH) * BDASH
    out_ref[0, 0, 0:T, :] = bsl_ref[0, 0]
    onehot = (lax.broadcasted_iota(jnp.int32, (1, BDASH), 1)
              == beam_k).astype(jnp.float32)
    out_ref[0, 0, T, :] = jax.lax.dot_general(
        onehot, lp_ref[0], (((1,), (0,)), ((), ())),
        preferred_element_type=jnp.float32)[0]


def kernel(logprobs, beam_logprobs_sum, beam_seq, beam_seq_logprobs, state,
           beam_size):
    del beam_size
    nrows = BATCH * BDASH

    new_bls3, new_seq, src3, new_state = pl.pallas_call(
        _topk_body,
        grid=(BATCH,),
        in_specs=[
            pl.BlockSpec((BDASH, VOCAB), lambda b: (b, 0)),
            pl.BlockSpec((1, BDASH, 1), lambda b: (b, 0, 0)),
            pl.BlockSpec((1, BDASH, T), lambda b: (b, 0, 0)),
            pl.BlockSpec((2, BDASH, DMODEL), lambda b: (0, b, 0)),
        ],
        out_specs=[
            pl.BlockSpec((1, 1, BDASH), lambda b: (b, 0, 0)),
            pl.BlockSpec((1, BDASH, T + 1), lambda b: (b, 0, 0)),
            pl.BlockSpec((1, 1, BDASH), lambda b: (b, 0, 0)),
            pl.BlockSpec((2, BDASH, DMODEL), lambda b: (0, b, 0)),
        ],
        out_shape=[
            jax.ShapeDtypeStruct((BATCH, 1, BDASH), jnp.float32),
            jax.ShapeDtypeStruct((BATCH, BDASH, T + 1), jnp.int32),
            jax.ShapeDtypeStruct((BATCH, 1, BDASH), jnp.int32),
            jax.ShapeDtypeStruct((2, nrows, DMODEL), jnp.float32),
        ],
    )(
        logprobs.reshape(nrows, VOCAB),
        beam_logprobs_sum.reshape(BATCH, BDASH, 1),
        beam_seq,
        state,
    )

    src_rows = src3.reshape(nrows)

    new_bsl = pl.pallas_call(
        _gather_body,
        grid_spec=pltpu.PrefetchScalarGridSpec(
            num_scalar_prefetch=1,
            grid=(nrows,),
            in_specs=[
                pl.BlockSpec((1, 1, T, VOCAB),
                             lambda i, sr: (sr[i] // BDASH, sr[i] % BDASH,
                                            0, 0)),
                pl.BlockSpec((1, BDASH, VOCAB),
                             lambda i, sr: (i // BDASH, 0, 0)),
            ],
            out_specs=[
                pl.BlockSpec((1, 1, T + 1, VOCAB),
                             lambda i, sr: (i // BDASH, i % BDASH, 0, 0)),
            ],
        ),
        out_shape=[
            jax.ShapeDtypeStruct((BATCH, BDASH, T + 1, VOCAB), jnp.float32),
        ],
    )(src_rows, beam_seq_logprobs, logprobs.reshape(BATCH, BDASH, VOCAB))[0]

    new_beam_logprobs_sum = new_bls3.reshape(BATCH, BDASH)
    return new_seq, new_bsl, new_beam_logprobs_sum, new_state


# speedup vs baseline: 3.9369x; 1.0886x over previous
import jax
import jax.numpy as jnp
from jax import lax
from jax.experimental import pallas as pl
from jax.experimental.pallas import tpu as pltpu

BATCH, BDASH, VOCAB, T, DMODEL = 16, 8, 32768, 4, 1024
NEG_INF = float("-inf")


def _topk_body(lp_ref, bls_ref, seq_ref, st_ref,
               bls_out, seq_out, src_out, st_out):
    b = pl.program_id(0)
    bls_col = bls_ref[0]
    cand = lp_ref[...] + bls_col
    fid = (lax.broadcasted_iota(jnp.int32, (BDASH, VOCAB), 0) * VOCAB
           + lax.broadcasted_iota(jnp.int32, (BDASH, VOCAB), 1))
    iota_col = lax.broadcasted_iota(jnp.int32, (BDASH, 1), 0)
    lane8 = lax.broadcasted_iota(jnp.int32, (1, 1, BDASH), 2)
    seq_slab = seq_ref[...]
    seq_mask_iota = lax.broadcasted_iota(jnp.int32, (1, BDASH, T), 1)
    st_blk = st_ref[...]
    st_iota = lax.broadcasted_iota(jnp.int32, (2, BDASH, DMODEL), 1)
    big = jnp.int32(2 ** 30)

    def emit(k, nb, sr, m, idx):
        beam_k = lax.shift_right_logical(idx, 15)
        sel_k = jnp.bitwise_and(idx, VOCAB - 1)
        blsv = jnp.sum(jnp.where(iota_col == beam_k, bls_col, 0.0))
        nb = jnp.where(lane8 == k, blsv + m, nb)
        sr = jnp.where(lane8 == k, b * BDASH + beam_k, sr)
        row = jnp.sum(jnp.where(seq_mask_iota == beam_k, seq_slab, 0),
                      axis=(0, 1))
        row5 = jnp.concatenate([row, jnp.broadcast_to(sel_k, (1,))], axis=0)
        seq_out[0, k, :] = row5
        st_out[:, k, :] = jnp.sum(
            jnp.where(st_iota == beam_k, st_blk, 0.0), axis=1)
        return nb, sr

    v, iv = cand, fid
    n = VOCAB
    while n > 128:
        h = n // 2
        a, bb = v[:, :h], v[:, h:]
        ge = a >= bb
        v = jnp.where(ge, a, bb)
        iv = jnp.where(ge, iv[:, :h], iv[:, h:])
        n = h

    nb = jnp.zeros((1, 1, BDASH), jnp.float32)
    sr = jnp.zeros((1, 1, BDASH), jnp.int32)
    ms, ids = [], []
    for k in range(BDASH):
        m = jnp.max(v)
        idx = jnp.min(jnp.where(v == m, iv, big))
        ms.append(m)
        ids.append(idx)
        v = jnp.where((v == m) & (iv == idx), NEG_INF, v)
        nb, sr = emit(k, nb, sr, m, idx)
    bls_out[...] = nb
    src_out[...] = sr

    v8, id8 = ms[BDASH - 1], ids[BDASH - 1]
    pred = (cand > v8) | ((cand == v8) & (fid <= id8))
    cnt = jnp.sum(pred.astype(jnp.int32))

    @pl.when(cnt != BDASH)
    def _slow_path():
        vals = cand
        nb = jnp.zeros((1, 1, BDASH), jnp.float32)
        sr = jnp.zeros((1, 1, BDASH), jnp.int32)
        for k in range(BDASH):
            m = jnp.max(vals)
            idx = jnp.min(jnp.where(vals == m, fid, big))
            nb, sr = emit(k, nb, sr, m, idx)
            vals = jnp.where(fid == idx, NEG_INF, vals)
        bls_out[...] = nb
        src_out[...] = sr


def _gather_body(src_ref, bsl_ref, lp_ref, out_ref):
    i = pl.program_id(0)
    beam_k = src_ref[i] - (i // BDASH) * BDASH
    out_ref[0, 0, 0:T, :] = bsl_ref[0, 0]
    sub_iota = lax.broadcasted_iota(jnp.int32, (BDASH, VOCAB), 0)
    out_ref[0, 0, T, :] = jnp.sum(
        jnp.where(sub_iota == beam_k, lp_ref[0], 0.0), axis=0)


def kernel(logprobs, beam_logprobs_sum, beam_seq, beam_seq_logprobs, state,
           beam_size):
    del beam_size
    nrows = BATCH * BDASH

    new_bls3, new_seq, src3, new_state = pl.pallas_call(
        _topk_body,
        grid=(BATCH,),
        in_specs=[
            pl.BlockSpec((BDASH, VOCAB), lambda b: (b, 0)),
            pl.BlockSpec((1, BDASH, 1), lambda b: (b, 0, 0)),
            pl.BlockSpec((1, BDASH, T), lambda b: (b, 0, 0)),
            pl.BlockSpec((2, BDASH, DMODEL), lambda b: (0, b, 0)),
        ],
        out_specs=[
            pl.BlockSpec((1, 1, BDASH), lambda b: (b, 0, 0)),
            pl.BlockSpec((1, BDASH, T + 1), lambda b: (b, 0, 0)),
            pl.BlockSpec((1, 1, BDASH), lambda b: (b, 0, 0)),
            pl.BlockSpec((2, BDASH, DMODEL), lambda b: (0, b, 0)),
        ],
        out_shape=[
            jax.ShapeDtypeStruct((BATCH, 1, BDASH), jnp.float32),
            jax.ShapeDtypeStruct((BATCH, BDASH, T + 1), jnp.int32),
            jax.ShapeDtypeStruct((BATCH, 1, BDASH), jnp.int32),
            jax.ShapeDtypeStruct((2, nrows, DMODEL), jnp.float32),
        ],
    )(
        logprobs.reshape(nrows, VOCAB),
        beam_logprobs_sum.reshape(BATCH, BDASH, 1),
        beam_seq,
        state,
    )

    src_rows = src3.reshape(nrows)

    new_bsl = pl.pallas_call(
        _gather_body,
        grid_spec=pltpu.PrefetchScalarGridSpec(
            num_scalar_prefetch=1,
            grid=(nrows,),
            in_specs=[
                pl.BlockSpec((1, 1, T, VOCAB),
                             lambda i, sr: (sr[i] // BDASH, sr[i] % BDASH,
                                            0, 0)),
                pl.BlockSpec((1, BDASH, VOCAB),
                             lambda i, sr: (i // BDASH, 0, 0)),
            ],
            out_specs=[
                pl.BlockSpec((1, 1, T + 1, VOCAB),
                             lambda i, sr: (i // BDASH, i % BDASH, 0, 0)),
            ],
        ),
        out_shape=[
            jax.ShapeDtypeStruct((BATCH, BDASH, T + 1, VOCAB), jnp.float32),
        ],
    )(src_rows, beam_seq_logprobs, logprobs.reshape(BATCH, BDASH, VOCAB))[0]

    new_beam_logprobs_sum = new_bls3.reshape(BATCH, BDASH)
    return new_seq, new_bsl, new_beam_logprobs_sum, new_state


# speedup vs baseline: 4.9110x; 1.2474x over previous
import jax
import jax.numpy as jnp
from jax import lax
from jax.experimental import pallas as pl
from jax.experimental.pallas import tpu as pltpu

BATCH, BDASH, VOCAB, T, DMODEL = 16, 8, 32768, 4, 1024
NEG_INF = float("-inf")


def _topk_body(lp_ref, bls_ref, seq_ref, st_ref,
               bls_out, seq_out, src_out, st_out):
    b = pl.program_id(0)
    bls_col = bls_ref[0]
    cand = lp_ref[...] + bls_col
    fid = (lax.broadcasted_iota(jnp.int32, (BDASH, VOCAB), 0) * VOCAB
           + lax.broadcasted_iota(jnp.int32, (BDASH, VOCAB), 1))
    iota_col = lax.broadcasted_iota(jnp.int32, (BDASH, 1), 0)
    lane8 = lax.broadcasted_iota(jnp.int32, (1, 1, BDASH), 2)
    seq_slab = seq_ref[...]
    seq_mask_iota = lax.broadcasted_iota(jnp.int32, (1, BDASH, T), 1)
    st_blk = st_ref[...]
    st_iota = lax.broadcasted_iota(jnp.int32, (2, BDASH, DMODEL), 1)
    big = jnp.int32(2 ** 30)

    def emit(k, nb, sr, m, idx):
        beam_k = lax.shift_right_logical(idx, 15)
        sel_k = jnp.bitwise_and(idx, VOCAB - 1)
        blsv = jnp.sum(jnp.where(iota_col == beam_k, bls_col, 0.0))
        nb = jnp.where(lane8 == k, blsv + m, nb)
        sr = jnp.where(lane8 == k, b * BDASH + beam_k, sr)
        row = jnp.sum(jnp.where(seq_mask_iota == beam_k, seq_slab, 0),
                      axis=(0, 1))
        row5 = jnp.concatenate([row, jnp.broadcast_to(sel_k, (1,))], axis=0)
        seq_out[0, k, :] = row5
        st_out[:, k, :] = jnp.sum(
            jnp.where(st_iota == beam_k, st_blk, 0.0), axis=1)
        return nb, sr

    v, iv = cand, fid
    n = VOCAB
    while n > 128:
        h = n // 2
        a, bb = v[:, :h], v[:, h:]
        ge = a >= bb
        v = jnp.where(ge, a, bb)
        iv = jnp.where(ge, iv[:, :h], iv[:, h:])
        n = h

    nb = jnp.zeros((1, 1, BDASH), jnp.float32)
    sr = jnp.zeros((1, 1, BDASH), jnp.int32)
    ms, ids = [], []
    for k in range(BDASH):
        m = jnp.max(v)
        idx = jnp.min(jnp.where(v == m, iv, big))
        ms.append(m)
        ids.append(idx)
        v = jnp.where((v == m) & (iv == idx), NEG_INF, v)
        nb, sr = emit(k, nb, sr, m, idx)
    bls_out[...] = nb
    src_out[...] = sr

    v8, id8 = ms[BDASH - 1], ids[BDASH - 1]
    pred = (cand > v8) | ((cand == v8) & (fid <= id8))
    cnt = jnp.sum(pred.astype(jnp.int32))

    @pl.when(cnt != BDASH)
    def _slow_path():
        vals = cand
        nb = jnp.zeros((1, 1, BDASH), jnp.float32)
        sr = jnp.zeros((1, 1, BDASH), jnp.int32)
        for k in range(BDASH):
            m = jnp.max(vals)
            idx = jnp.min(jnp.where(vals == m, fid, big))
            nb, sr = emit(k, nb, sr, m, idx)
            vals = jnp.where(fid == idx, NEG_INF, vals)
        bls_out[...] = nb
        src_out[...] = sr


def _gather_body(src_ref, bsl_ref, lp_ref, out_ref):
    b = pl.program_id(0)
    sub_iota = lax.broadcasted_iota(jnp.int32, (BDASH, VOCAB), 0)
    lp_blk = lp_ref[0]
    for k in range(BDASH):
        beam_k = src_ref[b * BDASH + k] - b * BDASH
        out_ref[0, k, 0:T, :] = bsl_ref[0, pl.ds(beam_k, 1)][0]
        out_ref[0, k, T, :] = jnp.sum(
            jnp.where(sub_iota == beam_k, lp_blk, 0.0), axis=0)


def kernel(logprobs, beam_logprobs_sum, beam_seq, beam_seq_logprobs, state,
           beam_size):
    del beam_size
    nrows = BATCH * BDASH

    new_bls3, new_seq, src3, new_state = pl.pallas_call(
        _topk_body,
        grid=(BATCH,),
        in_specs=[
            pl.BlockSpec((BDASH, VOCAB), lambda b: (b, 0)),
            pl.BlockSpec((1, BDASH, 1), lambda b: (b, 0, 0)),
            pl.BlockSpec((1, BDASH, T), lambda b: (b, 0, 0)),
            pl.BlockSpec((2, BDASH, DMODEL), lambda b: (0, b, 0)),
        ],
        out_specs=[
            pl.BlockSpec((1, 1, BDASH), lambda b: (b, 0, 0)),
            pl.BlockSpec((1, BDASH, T + 1), lambda b: (b, 0, 0)),
            pl.BlockSpec((1, 1, BDASH), lambda b: (b, 0, 0)),
            pl.BlockSpec((2, BDASH, DMODEL), lambda b: (0, b, 0)),
        ],
        out_shape=[
            jax.ShapeDtypeStruct((BATCH, 1, BDASH), jnp.float32),
            jax.ShapeDtypeStruct((BATCH, BDASH, T + 1), jnp.int32),
            jax.ShapeDtypeStruct((BATCH, 1, BDASH), jnp.int32),
            jax.ShapeDtypeStruct((2, nrows, DMODEL), jnp.float32),
        ],
    )(
        logprobs.reshape(nrows, VOCAB),
        beam_logprobs_sum.reshape(BATCH, BDASH, 1),
        beam_seq,
        state,
    )

    src_rows = src3.reshape(nrows)

    new_bsl = pl.pallas_call(
        _gather_body,
        grid_spec=pltpu.PrefetchScalarGridSpec(
            num_scalar_prefetch=1,
            grid=(BATCH,),
            in_specs=[
                pl.BlockSpec((1, BDASH, T, VOCAB),
                             lambda b, sr: (b, 0, 0, 0)),
                pl.BlockSpec((1, BDASH, VOCAB),
                             lambda b, sr: (b, 0, 0)),
            ],
            out_specs=[
                pl.BlockSpec((1, BDASH, T + 1, VOCAB),
                             lambda b, sr: (b, 0, 0, 0)),
            ],
        ),
        out_shape=[
            jax.ShapeDtypeStruct((BATCH, BDASH, T + 1, VOCAB), jnp.float32),
        ],
    )(src_rows, beam_seq_logprobs, logprobs.reshape(BATCH, BDASH, VOCAB))[0]

    new_beam_logprobs_sum = new_bls3.reshape(BATCH, BDASH)
    return new_seq, new_bsl, new_beam_logprobs_sum, new_state


# speedup vs baseline: 5.5112x; 1.1222x over previous
import jax
import jax.numpy as jnp
from jax import lax
from jax.experimental import pallas as pl
from jax.experimental.pallas import tpu as pltpu

BATCH, BDASH, VOCAB, T, DMODEL = 16, 8, 32768, 4, 1024
NEG_INF = float("-inf")


def _body(lp_ref, bls_ref, seq_ref, st_ref, bsl_hbm, lp_hbm,
          bls_out, seq_out, st_out, big_out, beam_smem, sem):
    b = pl.program_id(0)
    bls_col = bls_ref[0]
    cand = lp_ref[...] + bls_col
    fid = (lax.broadcasted_iota(jnp.int32, (BDASH, VOCAB), 0) * VOCAB
           + lax.broadcasted_iota(jnp.int32, (BDASH, VOCAB), 1))
    iota_col = lax.broadcasted_iota(jnp.int32, (BDASH, 1), 0)
    lane8 = lax.broadcasted_iota(jnp.int32, (1, 1, BDASH), 2)
    seq_slab = seq_ref[...]
    seq_mask_iota = lax.broadcasted_iota(jnp.int32, (1, BDASH, T), 1)
    st_blk = st_ref[...]
    st_iota = lax.broadcasted_iota(jnp.int32, (2, BDASH, DMODEL), 1)
    big = jnp.int32(2 ** 30)

    def emit(k, nb, m, idx):
        beam_k = lax.shift_right_logical(idx, 15)
        sel_k = jnp.bitwise_and(idx, VOCAB - 1)
        beam_smem[k] = beam_k
        blsv = jnp.sum(jnp.where(iota_col == beam_k, bls_col, 0.0))
        nb = jnp.where(lane8 == k, blsv + m, nb)
        row = jnp.sum(jnp.where(seq_mask_iota == beam_k, seq_slab, 0),
                      axis=(0, 1))
        row5 = jnp.concatenate([row, jnp.broadcast_to(sel_k, (1,))], axis=0)
        seq_out[0, k, :] = row5
        st_out[:, k, :] = jnp.sum(
            jnp.where(st_iota == beam_k, st_blk, 0.0), axis=1)
        return nb

    v, iv = cand, fid
    n = VOCAB
    while n > 128:
        h = n // 2
        a, bb = v[:, :h], v[:, h:]
        ge = a >= bb
        v = jnp.where(ge, a, bb)
        iv = jnp.where(ge, iv[:, :h], iv[:, h:])
        n = h

    nb = jnp.zeros((1, 1, BDASH), jnp.float32)
    ms, ids = [], []
    for k in range(BDASH):
        m = jnp.max(v)
        idx = jnp.min(jnp.where(v == m, iv, big))
        ms.append(m)
        ids.append(idx)
        v = jnp.where((v == m) & (iv == idx), NEG_INF, v)
        nb = emit(k, nb, m, idx)
    bls_out[...] = nb

    v8, id8 = ms[BDASH - 1], ids[BDASH - 1]
    pred = (cand > v8) | ((cand == v8) & (fid <= id8))
    cnt = jnp.sum(pred.astype(jnp.int32))

    @pl.when(cnt != BDASH)
    def _slow_path():
        vals = cand
        nb = jnp.zeros((1, 1, BDASH), jnp.float32)
        for k in range(BDASH):
            m = jnp.max(vals)
            idx = jnp.min(jnp.where(vals == m, fid, big))
            nb = emit(k, nb, m, idx)
            vals = jnp.where(fid == idx, NEG_INF, vals)
        bls_out[...] = nb

    copies = []
    for k in range(BDASH):
        src = b * BDASH + beam_smem[k]
        copies.append(pltpu.make_async_copy(
            bsl_hbm.at[src], big_out.at[0, k, 0:T, :], sem))
        copies.append(pltpu.make_async_copy(
            lp_hbm.at[pl.ds(src, 1), :], big_out.at[0, k, pl.ds(T, 1), :],
            sem))
    for c in copies:
        c.start()
    for c in copies:
        c.wait()


def kernel(logprobs, beam_logprobs_sum, beam_seq, beam_seq_logprobs, state,
           beam_size):
    del beam_size
    nrows = BATCH * BDASH
    lp2 = logprobs.reshape(nrows, VOCAB)

    new_bls3, new_seq, new_state, new_bsl = pl.pallas_call(
        _body,
        grid=(BATCH,),
        in_specs=[
            pl.BlockSpec((BDASH, VOCAB), lambda b: (b, 0)),
            pl.BlockSpec((1, BDASH, 1), lambda b: (b, 0, 0)),
            pl.BlockSpec((1, BDASH, T), lambda b: (b, 0, 0)),
            pl.BlockSpec((2, BDASH, DMODEL), lambda b: (0, b, 0)),
            pl.BlockSpec(memory_space=pltpu.MemorySpace.HBM),
            pl.BlockSpec(memory_space=pltpu.MemorySpace.HBM),
        ],
        out_specs=[
            pl.BlockSpec((1, 1, BDASH), lambda b: (b, 0, 0)),
            pl.BlockSpec((1, BDASH, T + 1), lambda b: (b, 0, 0)),
            pl.BlockSpec((2, BDASH, DMODEL), lambda b: (0, b, 0)),
            pl.BlockSpec((1, BDASH, T + 1, VOCAB), lambda b: (b, 0, 0, 0)),
        ],
        out_shape=[
            jax.ShapeDtypeStruct((BATCH, 1, BDASH), jnp.float32),
            jax.ShapeDtypeStruct((BATCH, BDASH, T + 1), jnp.int32),
            jax.ShapeDtypeStruct((2, nrows, DMODEL), jnp.float32),
            jax.ShapeDtypeStruct((BATCH, BDASH, T + 1, VOCAB), jnp.float32),
        ],
        scratch_shapes=[
            pltpu.SMEM((BDASH,), jnp.int32),
            pltpu.SemaphoreType.DMA,
        ],
    )(
        lp2,
        beam_logprobs_sum.reshape(BATCH, BDASH, 1),
        beam_seq,
        state,
        beam_seq_logprobs.reshape(nrows, T, VOCAB),
        lp2,
    )

    new_beam_logprobs_sum = new_bls3.reshape(BATCH, BDASH)
    return new_seq, new_bsl, new_beam_logprobs_sum, new_state
